# Initial kernel scaffold; baseline (speedup 1.0000x reference)
import jax
import jax.numpy as jnp
from jax.experimental import pallas as pl


def kernel(x, attr, mus, sigmas):
    raise NotImplementedError("write your pallas kernel here")


# speedup vs baseline: 1.8310x; 1.8310x over previous
import functools

import jax
import jax.numpy as jnp
from jax import lax
from jax.experimental import pallas as pl
from jax.experimental.pallas import tpu as pltpu
from jax.experimental.pallas import tpu_sc as plsc

_EPS = 1e-6
_L = 16
_NC = 2
_NS = 16
_NW = _NC * _NS


def _softplus16(s):
    t = jnp.exp(-jnp.abs(s))
    c = 1.0 + t
    y = t * (0.9991150 + t * (-0.4899597 + t * 0.1560245))
    y = y - 1.0 + c * jnp.exp(-y)
    y = y - 1.0 + c * jnp.exp(-y)
    return jnp.maximum(s, 0.0) + y


def kernel(x, attr, mus, sigmas):
    B, D = x.shape
    A = mus.shape[0]
    G = D // _L
    rows_w = B // _NW
    CH = min(128, rows_w)
    nch = rows_w // CH

    mesh = plsc.VectorSubcoreMesh(core_axis_name="c", subcore_axis_name="s")

    @functools.partial(
        pl.kernel,
        out_type=jax.ShapeDtypeStruct((B, D), jnp.float32),
        mesh=mesh,
        compiler_params=pltpu.CompilerParams(needs_layout_passes=False),
        scratch_types=[
            pltpu.VMEM((A, D), jnp.float32),
            pltpu.VMEM((A, D), jnp.float32),
            pltpu.VMEM((A * D,), jnp.float32),
            pltpu.VMEM((A * D,), jnp.float32),
            pltpu.VMEM((CH, D), jnp.float32),
            pltpu.VMEM((CH, D), jnp.float32),
            pltpu.VMEM((CH,), jnp.int32),
        ],
    )
    def sc_kernel(x_hbm, attr_hbm, mus_hbm, sig_hbm, out_hbm,
                  mus_v, sig_v, scale_v, bias_v, xb, ob, ab):
        wid = lax.axis_index("s") * _NC + lax.axis_index("c")
        base = wid * rows_w

        pltpu.sync_copy(mus_hbm, mus_v)
        pltpu.sync_copy(sig_hbm, sig_v)

        iota = lax.iota(jnp.int32, _L)
        for r in range(A):
            for g in range(G):
                s = sig_v[r, pl.ds(g * _L, _L)]
                m = mus_v[r, pl.ds(g * _L, _L)]
                sc = 1.0 / (_softplus16(s) + _EPS)
                fo = (r * G + g) * _L
                scale_v[pl.ds(fo, _L)] = sc
                bias_v[pl.ds(fo, _L)] = -m * sc

        offs = [iota + g * _L for g in range(G)]

        for t in range(nch):
            r0 = base + t * CH
            pltpu.sync_copy(x_hbm.at[pl.ds(r0, CH), :], xb)
            pltpu.sync_copy(attr_hbm.at[pl.ds(r0, CH)], ab)

            def row_body(jg, carry):
                av = ab[pl.ds(jg * _L, _L)] * D
                for l in range(_L):
                    j = jg * _L + l
                    bvec = jnp.full((_L,), av[l], jnp.int32)
                    for g in range(G):
                        idx = bvec + offs[g]
                        scv = plsc.load_gather(scale_v, [idx])
                        bsv = plsc.load_gather(bias_v, [idx])
                        xv = xb[j, pl.ds(g * _L, _L)]
                        ob[j, pl.ds(g * _L, _L)] = xv * scv + bsv
                return carry

            lax.fori_loop(0, CH // _L, row_body, 0)
            pltpu.sync_copy(ob, out_hbm.at[pl.ds(r0, CH), :])

    return sc_kernel(x, attr, mus, sigmas)


# speedup vs baseline: 2.1006x; 1.1473x over previous
import functools

import jax
import jax.numpy as jnp
from jax import lax
from jax.experimental import pallas as pl
from jax.experimental.pallas import tpu as pltpu
from jax.experimental.pallas import tpu_sc as plsc

_EPS = 1e-6
_L = 16
_NC = 2
_NS = 16
_NW = _NC * _NS


def _softplus16(s):
    t = jnp.exp(-jnp.abs(s))
    c = 1.0 + t
    y = t * (0.9991150 + t * (-0.4899597 + t * 0.1560245))
    y = y - 1.0 + c * jnp.exp(-y)
    y = y - 1.0 + c * jnp.exp(-y)
    return jnp.maximum(s, 0.0) + y


def kernel(x, attr, mus, sigmas):
    B, D = x.shape
    A = mus.shape[0]
    G = D // _L
    rows_w = B // _NW
    CH = min(128, rows_w)
    nch = rows_w // CH

    mesh = plsc.VectorSubcoreMesh(core_axis_name="c", subcore_axis_name="s")

    @functools.partial(
        pl.kernel,
        out_type=jax.ShapeDtypeStruct((B, D), jnp.float32),
        mesh=mesh,
        compiler_params=pltpu.CompilerParams(needs_layout_passes=False),
        scratch_types=[
            pltpu.VMEM((A, D), jnp.float32),
            pltpu.VMEM((A, D), jnp.float32),
            pltpu.VMEM((A * D,), jnp.float32),
            pltpu.VMEM((A * D,), jnp.float32),
            pltpu.VMEM((2, CH, D), jnp.float32),
            pltpu.VMEM((2, CH, D), jnp.float32),
            pltpu.VMEM((2, CH), jnp.int32),
            pltpu.SemaphoreType.DMA,
            pltpu.SemaphoreType.DMA,
            pltpu.SemaphoreType.DMA,
            pltpu.SemaphoreType.DMA,
        ],
    )
    def sc_kernel(x_hbm, attr_hbm, mus_hbm, sig_hbm, out_hbm,
                  mus_v, sig_v, scale_v, bias_v, xb2, ob2, ab2,
                  isem0, isem1, osem0, osem1):
        isems = (isem0, isem1)
        osems = (osem0, osem1)
        wid = lax.axis_index("s") * _NC + lax.axis_index("c")
        base = wid * rows_w

        def start_in(t):
            b = t % 2
            r0 = base + t * CH
            dx = pltpu.async_copy(x_hbm.at[pl.ds(r0, CH), :], xb2.at[b], isems[b])
            da = pltpu.async_copy(attr_hbm.at[pl.ds(r0, CH)], ab2.at[b], isems[b])
            return (dx, da)

        in_desc = {0: start_in(0)}

        pltpu.sync_copy(mus_hbm, mus_v)
        pltpu.sync_copy(sig_hbm, sig_v)

        iota = lax.iota(jnp.int32, _L)
        for r in range(A):
            for g in range(G):
                s = sig_v[r, pl.ds(g * _L, _L)]
                m = mus_v[r, pl.ds(g * _L, _L)]
                sc = 1.0 / (_softplus16(s) + _EPS)
                fo = (r * G + g) * _L
                scale_v[pl.ds(fo, _L)] = sc
                bias_v[pl.ds(fo, _L)] = -m * sc

        offs = [iota + g * _L for g in range(G)]

        out_desc = {}
        for t in range(nch):
            b = t % 2
            if t + 1 < nch:
                in_desc[t + 1] = start_in(t + 1)
            for d in in_desc.pop(t):
                d.wait()
            if t - 2 in out_desc:
                out_desc.pop(t - 2).wait()
            xb, ob, ab = xb2.at[b], ob2.at[b], ab2.at[b]

            def row_body(jg, carry):
                av = ab[pl.ds(jg * _L, _L)] * D
                for l in range(_L):
                    j = jg * _L + l
                    bvec = jnp.full((_L,), av[l], jnp.int32)
                    for g in range(G):
                        idx = bvec + offs[g]
                        scv = plsc.load_gather(scale_v, [idx])
                        bsv = plsc.load_gather(bias_v, [idx])
                        xv = xb[j, pl.ds(g * _L, _L)]
                        ob[j, pl.ds(g * _L, _L)] = xv * scv + bsv
                return carry

            lax.fori_loop(0, CH // _L, row_body, 0)
            r0 = base + t * CH
            out_desc[t] = pltpu.async_copy(
                ob, out_hbm.at[pl.ds(r0, CH), :], osems[b])
        for t in sorted(out_desc):
            out_desc.pop(t).wait()

    return sc_kernel(x, attr, mus, sigmas)
